# baseline (device time: 188873 ns/iter reference)
import jax
import jax.numpy as jnp
from jax import lax
from jax.experimental import pallas as pl
from jax.experimental.pallas import tpu as pltpu

N_DEV = 4
M = 512
K = 1024
NL = 8192
NG = N_DEV * NL
TW = 1024
NT = NL // TW
TN = 2048
MH = M // 2


def _gather_normalize(x, W):
    def body(x_ref, w_hbm, probs_ref, lgopp_ref, m_ref, r_ref,
             comm_ref, wt_ref, stats_ref, stage_ref,
             w_sems, send_sems, recv_sems, s2_send, s2_recv,
             st_send, st_recv, stage_sems, lg_sem):
        my = lax.axis_index("i")
        left = lax.rem(my + (N_DEV - 1), N_DEV)
        right = lax.rem(my + 1, N_DEV)

        barrier_sem = pltpu.get_barrier_semaphore()
        for nbr in (left, right):
            pl.semaphore_signal(
                barrier_sem, inc=1,
                device_id=(nbr,), device_id_type=pl.DeviceIdType.MESH,
            )
        pl.semaphore_wait(barrier_sem, 2)

        def remote_copy(src, dst, sends, recvs, idx, dst_dev):
            return pltpu.make_async_remote_copy(
                src_ref=src,
                dst_ref=dst,
                send_sem=sends.at[idx],
                recv_sem=recvs.at[idx],
                device_id=(dst_dev,),
                device_id_type=pl.DeviceIdType.MESH,
            )

        def cols(origin):
            return pl.ds(pl.multiple_of(origin * NL, NL), NL)

        def sub_cols(origin, j):
            return pl.ds(pl.multiple_of(origin * NL + j * TW, TW), TW)

        def w_dma(j):
            return pltpu.make_async_copy(
                w_hbm.at[:, pl.ds(j * TW, TW)],
                wt_ref.at[j % 2],
                w_sems.at[j % 2],
            )

        w_dma(0).start()
        xb = x_ref[...].astype(jnp.bfloat16)
        hop1 = []
        m0 = None
        for j in range(NT):
            if j + 1 < NT:
                w_dma(j + 1).start()
            w_dma(j).wait()
            wb = wt_ref[j % 2].astype(jnp.bfloat16)
            lt = jnp.dot(xb, wb, preferred_element_type=jnp.float32)
            comm_ref[:, sub_cols(my, j)] = lt.astype(jnp.bfloat16)
            tm = jnp.max(lt, axis=1, keepdims=True)
            m0 = tm if m0 is None else jnp.maximum(m0, tm)
            r1j = remote_copy(
                comm_ref.at[:, sub_cols(my, j)],
                comm_ref.at[:, sub_cols(my, j)],
                send_sems, recv_sems, j, right,
            )
            l1j = remote_copy(
                comm_ref.at[:, sub_cols(my, j)],
                comm_ref.at[:, sub_cols(my, j)],
                send_sems, recv_sems, NT + j, left,
            )
            r1j.start()
            l1j.start()
            hop1.append((r1j, l1j))

        def s0_body(t, s):
            l = comm_ref[:, pl.ds(my * NL + t * TN, TN)].astype(jnp.float32)
            return s + jnp.sum(jnp.exp(l - m0), axis=1, keepdims=True)

        s0 = lax.fori_loop(
            0, NL // TN, s0_body, jnp.zeros((M, 1), jnp.float32)
        )
        stats_ref[0] = jnp.concatenate(
            [jnp.broadcast_to(m0, (M, 4)), jnp.broadcast_to(s0, (M, 4))],
            axis=1,
        )
        st_r1 = remote_copy(
            stats_ref.at[0], stats_ref.at[1], st_send, st_recv, 0, right
        )
        st_l1 = remote_copy(
            stats_ref.at[0], stats_ref.at[2], st_send, st_recv, 1, left
        )
        st_r1.start()
        st_l1.start()

        for r1j, _ in hop1:
            r1j.wait_recv()
        st_r1.wait_recv()
        st_r2 = remote_copy(
            stats_ref.at[1], stats_ref.at[3], st_send, st_recv, 2, right
        )
        st_r2.start()
        r2 = remote_copy(
            comm_ref.at[pl.ds(0, MH), cols(left)],
            comm_ref.at[pl.ds(0, MH), cols(left)],
            s2_send, s2_recv, 0, right,
        )
        r2.start()
        for _, l1j in hop1:
            l1j.wait_recv()
        l2 = remote_copy(
            comm_ref.at[pl.ds(MH, MH), cols(right)],
            comm_ref.at[pl.ds(MH, MH), cols(right)],
            s2_send, s2_recv, 1, left,
        )
        l2.start()
        for r1j, l1j in hop1:
            r1j.wait_send()
            l1j.wait_send()
        st_l1.wait_recv()
        st_r2.wait_recv()

        mc = [stats_ref[c, :, 0:1] for c in range(N_DEV)]
        sc = [stats_ref[c, :, 4:5] for c in range(N_DEV)]
        m_g = jnp.maximum(jnp.maximum(mc[0], mc[1]),
                          jnp.maximum(mc[2], mc[3]))
        s_g = sum(s * jnp.exp(m - m_g) for s, m in zip(sc, mc))
        m_ref[...] = jnp.broadcast_to(m_g, (M, 128))
        r_ref[...] = jnp.broadcast_to(1.0 / s_g, (M, 128))
        m_bf = m_g.astype(jnp.bfloat16)
        r_bf = (1.0 / s_g).astype(jnp.bfloat16)

        out_dmas = []
        for origin in (my, left, right):
            for t in range(NL // TN):
                i = len(out_dmas)
                slot = i % 2
                if i >= 2:
                    out_dmas[i - 2].wait()
                c0 = origin * NL + t * TN
                p = jnp.exp(comm_ref[:, pl.ds(c0, TN)] - m_bf) * r_bf
                stage_ref[slot] = p
                dma = pltpu.make_async_copy(
                    stage_ref.at[slot],
                    probs_ref.at[:, pl.ds(c0, TN)],
                    stage_sems.at[slot],
                )
                dma.start()
                out_dmas.append(dma)

        r2.wait_recv()
        l2.wait_recv()
        r2.wait_send()
        l2.wait_send()
        st_r1.wait_send()
        st_l1.wait_send()
        st_r2.wait_send()
        opp = lax.rem(my + 2, N_DEV)
        lg_dma = pltpu.make_async_copy(
            comm_ref.at[:, cols(opp)], lgopp_ref, lg_sem
        )
        lg_dma.start()
        for dma in out_dmas[-2:]:
            dma.wait()
        lg_dma.wait()

    return pl.pallas_call(
        body,
        out_shape=[
            jax.ShapeDtypeStruct((M, NG), jnp.bfloat16),
            jax.ShapeDtypeStruct((M, NL), jnp.bfloat16),
            jax.ShapeDtypeStruct((M, 128), jnp.float32),
            jax.ShapeDtypeStruct((M, 128), jnp.float32),
        ],
        in_specs=[
            pl.BlockSpec(memory_space=pltpu.VMEM),
            pl.BlockSpec(memory_space=pl.ANY),
        ],
        out_specs=[
            pl.BlockSpec(memory_space=pl.ANY),
            pl.BlockSpec(memory_space=pl.ANY),
            pl.BlockSpec(memory_space=pltpu.VMEM),
            pl.BlockSpec(memory_space=pltpu.VMEM),
        ],
        scratch_shapes=[
            pltpu.VMEM((M, NG), jnp.bfloat16),
            pltpu.VMEM((2, K, TW), jnp.float32),
            pltpu.VMEM((N_DEV, M, 8), jnp.float32),
            pltpu.VMEM((2, M, TN), jnp.bfloat16),
            pltpu.SemaphoreType.DMA((2,)),
            pltpu.SemaphoreType.DMA((2 * NT,)),
            pltpu.SemaphoreType.DMA((2 * NT,)),
            pltpu.SemaphoreType.DMA((2,)),
            pltpu.SemaphoreType.DMA((2,)),
            pltpu.SemaphoreType.DMA((3,)),
            pltpu.SemaphoreType.DMA((3,)),
            pltpu.SemaphoreType.DMA((2,)),
            pltpu.SemaphoreType.DMA,
        ],
        compiler_params=pltpu.CompilerParams(
            collective_id=0, vmem_limit_bytes=60 * 1024 * 1024
        ),
    )(x, W)


def _finish(probs, lgopp, m, r):
    def body(lgopp_ref, probs_in, m_ref, r_ref, out_ref,
             tile_ref, stage_ref, in_sems, out_sems):
        del probs_in
        my = lax.axis_index("i")
        opp = lax.rem(my + 2, N_DEV)
        m_bf = m_ref[:, 0:1].astype(jnp.bfloat16)
        r_bf = r_ref[:, 0:1].astype(jnp.bfloat16)

        def in_dma(t):
            return pltpu.make_async_copy(
                lgopp_ref.at[:, pl.ds(t * TN, TN)],
                tile_ref.at[t % 2],
                in_sems.at[t % 2],
            )

        def out_dma(t):
            return pltpu.make_async_copy(
                stage_ref.at[t % 2],
                out_ref.at[:, pl.ds(
                    pl.multiple_of(opp * NL, NL) + t * TN, TN)],
                out_sems.at[t % 2],
            )

        n = NL // TN
        in_dma(0).start()
        for t in range(n):
            if t + 1 < n:
                in_dma(t + 1).start()
            in_dma(t).wait()
            if t >= 2:
                out_dma(t - 2).wait()
            stage_ref[t % 2] = (
                jnp.exp(tile_ref[t % 2] - m_bf) * r_bf
            )
            out_dma(t).start()
        for t in range(max(0, n - 2), n):
            out_dma(t).wait()

    return pl.pallas_call(
        body,
        out_shape=jax.ShapeDtypeStruct((M, NG), jnp.bfloat16),
        in_specs=[
            pl.BlockSpec(memory_space=pl.ANY),
            pl.BlockSpec(memory_space=pl.ANY),
            pl.BlockSpec(memory_space=pltpu.VMEM),
            pl.BlockSpec(memory_space=pltpu.VMEM),
        ],
        out_specs=pl.BlockSpec(memory_space=pl.ANY),
        input_output_aliases={1: 0},
        scratch_shapes=[
            pltpu.VMEM((2, M, TN), jnp.bfloat16),
            pltpu.VMEM((2, M, TN), jnp.bfloat16),
            pltpu.SemaphoreType.DMA((2,)),
            pltpu.SemaphoreType.DMA((2,)),
        ],
    )(lgopp, probs, m, r)


def kernel(x, W):
    probs, lgopp, m, r = _gather_normalize(x, W)
    return _finish(probs, lgopp, m, r)


# device time: 180898 ns/iter; 1.0441x vs baseline; 1.0441x over previous
import jax
import jax.numpy as jnp
from jax import lax
from jax.experimental import pallas as pl
from jax.experimental.pallas import tpu as pltpu

N_DEV = 4
M = 512
K = 1024
NL = 8192
NG = N_DEV * NL
TW = 1024
NT = NL // TW
TN = 2048
MH = M // 2


def _gather_normalize(x, W):
    def body(x_ref, w_hbm, probs_ref, m_ref, r_ref,
             comm_ref, wt_ref, stats_ref, stage_ref,
             w_sems, send_sems, recv_sems, s2_send, s2_recv,
             st_send, st_recv, stage_sems, lg_sem):
        my = lax.axis_index("i")
        left = lax.rem(my + (N_DEV - 1), N_DEV)
        right = lax.rem(my + 1, N_DEV)

        barrier_sem = pltpu.get_barrier_semaphore()
        for nbr in (left, right):
            pl.semaphore_signal(
                barrier_sem, inc=1,
                device_id=(nbr,), device_id_type=pl.DeviceIdType.MESH,
            )
        pl.semaphore_wait(barrier_sem, 2)

        def remote_copy(src, dst, sends, recvs, idx, dst_dev):
            return pltpu.make_async_remote_copy(
                src_ref=src,
                dst_ref=dst,
                send_sem=sends.at[idx],
                recv_sem=recvs.at[idx],
                device_id=(dst_dev,),
                device_id_type=pl.DeviceIdType.MESH,
            )

        def cols(origin):
            return pl.ds(pl.multiple_of(origin * NL, NL), NL)

        def sub_cols(origin, j):
            return pl.ds(pl.multiple_of(origin * NL + j * TW, TW), TW)

        def w_dma(j):
            return pltpu.make_async_copy(
                w_hbm.at[:, pl.ds(j * TW, TW)],
                wt_ref.at[j % 2],
                w_sems.at[j % 2],
            )

        w_dma(0).start()
        xb = x_ref[...].astype(jnp.bfloat16)
        hop1 = []
        m0 = None
        for j in range(NT):
            if j + 1 < NT:
                w_dma(j + 1).start()
            w_dma(j).wait()
            wb = wt_ref[j % 2].astype(jnp.bfloat16)
            lt = jnp.dot(xb, wb, preferred_element_type=jnp.float32)
            comm_ref[:, sub_cols(my, j)] = lt.astype(jnp.bfloat16)
            tm = jnp.max(lt, axis=1, keepdims=True)
            m0 = tm if m0 is None else jnp.maximum(m0, tm)
            r1j = remote_copy(
                comm_ref.at[:, sub_cols(my, j)],
                comm_ref.at[:, sub_cols(my, j)],
                send_sems, recv_sems, j, right,
            )
            l1j = remote_copy(
                comm_ref.at[:, sub_cols(my, j)],
                comm_ref.at[:, sub_cols(my, j)],
                send_sems, recv_sems, NT + j, left,
            )
            r1j.start()
            l1j.start()
            hop1.append((r1j, l1j))

        def s0_body(t, s):
            l = comm_ref[:, pl.ds(my * NL + t * TN, TN)].astype(jnp.float32)
            return s + jnp.sum(jnp.exp(l - m0), axis=1, keepdims=True)

        s0 = lax.fori_loop(
            0, NL // TN, s0_body, jnp.zeros((M, 1), jnp.float32)
        )
        stats_ref[0] = jnp.concatenate(
            [jnp.broadcast_to(m0, (M, 4)), jnp.broadcast_to(s0, (M, 4))],
            axis=1,
        )
        st_r1 = remote_copy(
            stats_ref.at[0], stats_ref.at[1], st_send, st_recv, 0, right
        )
        st_l1 = remote_copy(
            stats_ref.at[0], stats_ref.at[2], st_send, st_recv, 1, left
        )
        st_r1.start()
        st_l1.start()

        for r1j, _ in hop1:
            r1j.wait_recv()
        st_r1.wait_recv()
        st_r2 = remote_copy(
            stats_ref.at[1], stats_ref.at[3], st_send, st_recv, 2, right
        )
        st_r2.start()
        r2 = remote_copy(
            comm_ref.at[pl.ds(0, MH), cols(left)],
            comm_ref.at[pl.ds(0, MH), cols(left)],
            s2_send, s2_recv, 0, right,
        )
        r2.start()
        for _, l1j in hop1:
            l1j.wait_recv()
        l2 = remote_copy(
            comm_ref.at[pl.ds(MH, MH), cols(right)],
            comm_ref.at[pl.ds(MH, MH), cols(right)],
            s2_send, s2_recv, 1, left,
        )
        l2.start()
        for r1j, l1j in hop1:
            r1j.wait_send()
            l1j.wait_send()
        st_l1.wait_recv()
        st_r2.wait_recv()

        mc = [stats_ref[c, :, 0:1] for c in range(N_DEV)]
        sc = [stats_ref[c, :, 4:5] for c in range(N_DEV)]
        m_g = jnp.maximum(jnp.maximum(mc[0], mc[1]),
                          jnp.maximum(mc[2], mc[3]))
        s_g = sum(s * jnp.exp(m - m_g) for s, m in zip(sc, mc))
        m_ref[...] = jnp.broadcast_to(m_g, (M, 128))
        r_ref[...] = jnp.broadcast_to(1.0 / s_g, (M, 128))
        m_bf = m_g.astype(jnp.bfloat16)
        r_bf = (1.0 / s_g).astype(jnp.bfloat16)

        out_dmas = []
        for origin in (my, left, right):
            for t in range(NL // TN):
                i = len(out_dmas)
                slot = i % 2
                if i >= 2:
                    out_dmas[i - 2].wait()
                c0 = origin * NL + t * TN
                p = jnp.exp(comm_ref[:, pl.ds(c0, TN)] - m_bf) * r_bf
                stage_ref[slot] = p
                dma = pltpu.make_async_copy(
                    stage_ref.at[slot],
                    probs_ref.at[:, pl.ds(c0, TN)],
                    stage_sems.at[slot],
                )
                dma.start()
                out_dmas.append(dma)

        r2.wait_recv()
        l2.wait_recv()
        r2.wait_send()
        l2.wait_send()
        st_r1.wait_send()
        st_l1.wait_send()
        st_r2.wait_send()
        opp = lax.rem(my + 2, N_DEV)
        lg_dma = pltpu.make_async_copy(
            comm_ref.at[:, cols(opp)], probs_ref.at[:, cols(opp)], lg_sem
        )
        lg_dma.start()
        for dma in out_dmas[-2:]:
            dma.wait()
        lg_dma.wait()

    return pl.pallas_call(
        body,
        out_shape=[
            jax.ShapeDtypeStruct((M, NG), jnp.bfloat16),
            jax.ShapeDtypeStruct((M, 128), jnp.float32),
            jax.ShapeDtypeStruct((M, 128), jnp.float32),
        ],
        in_specs=[
            pl.BlockSpec(memory_space=pltpu.VMEM),
            pl.BlockSpec(memory_space=pl.ANY),
        ],
        out_specs=[
            pl.BlockSpec(memory_space=pl.ANY),
            pl.BlockSpec(memory_space=pltpu.VMEM),
            pl.BlockSpec(memory_space=pltpu.VMEM),
        ],
        scratch_shapes=[
            pltpu.VMEM((M, NG), jnp.bfloat16),
            pltpu.VMEM((2, K, TW), jnp.float32),
            pltpu.VMEM((N_DEV, M, 8), jnp.float32),
            pltpu.VMEM((2, M, TN), jnp.bfloat16),
            pltpu.SemaphoreType.DMA((2,)),
            pltpu.SemaphoreType.DMA((2 * NT,)),
            pltpu.SemaphoreType.DMA((2 * NT,)),
            pltpu.SemaphoreType.DMA((2,)),
            pltpu.SemaphoreType.DMA((2,)),
            pltpu.SemaphoreType.DMA((3,)),
            pltpu.SemaphoreType.DMA((3,)),
            pltpu.SemaphoreType.DMA((2,)),
            pltpu.SemaphoreType.DMA,
        ],
        compiler_params=pltpu.CompilerParams(
            collective_id=0, vmem_limit_bytes=60 * 1024 * 1024
        ),
    )(x, W)


def _finish(probs, m, r):
    TB = 4096
    blocks_per_chunk = NL // TB

    def body(src_ref, m_ref, r_ref, out_ref):
        my = lax.axis_index("i")
        opp = lax.rem(my + 2, N_DEV)
        t = pl.program_id(0)

        @pl.when(t // blocks_per_chunk == opp)
        def _():
            m_bf = m_ref[:, 0:1].astype(jnp.bfloat16)
            r_bf = r_ref[:, 0:1].astype(jnp.bfloat16)
            out_ref[...] = jnp.exp(src_ref[...] - m_bf) * r_bf

        @pl.when(t // blocks_per_chunk != opp)
        def _():
            out_ref[...] = src_ref[...]

    return pl.pallas_call(
        body,
        grid=(NG // TB,),
        in_specs=[
            pl.BlockSpec((M, TB), lambda t: (0, t)),
            pl.BlockSpec((M, 128), lambda t: (0, 0)),
            pl.BlockSpec((M, 128), lambda t: (0, 0)),
        ],
        out_specs=pl.BlockSpec((M, TB), lambda t: (0, t)),
        out_shape=jax.ShapeDtypeStruct((M, NG), jnp.bfloat16),
    )(probs, m, r)


def kernel(x, W):
    probs, m, r = _gather_normalize(x, W)
    return _finish(probs, m, r)
